# baseline (speedup 1.0000x reference)
import jax
import jax.numpy as jnp
from jax.experimental import pallas as pl
from jax.experimental.pallas import tpu as pltpu
from jax.experimental.pallas import tpu_sc as plsc

EMBED_DIM = 64
WINDOW = 128
SUBGATHERS = 2


def kernel(token_ids, embedding):
    batch, hist = token_ids.shape
    n_idx = batch * hist
    step_rows = WINDOW * SUBGATHERS
    vocab = embedding.shape[0]

    idx = token_ids.reshape(-1).astype(jnp.int32)
    rows = (idx >> 1).reshape(n_idx // WINDOW, WINDOW)
    table2 = embedding.reshape(vocab // 2, 2 * EMBED_DIM)

    mesh = plsc.VectorSubcoreMesh(core_axis_name="core", subcore_axis_name="subcore")

    @pl.kernel(
        out_type=jax.ShapeDtypeStruct((n_idx, 2 * EMBED_DIM), embedding.dtype),
        mesh=mesh,
        scratch_types=[pltpu.SemaphoreType.DMA],
    )
    def gather_kernel(table_hbm, idx_hbm, out_hbm, sem):
        def body(i_vmem, o_vmem):
            copies = [
                pltpu.async_copy(
                    table_hbm.at[i_vmem.at[j]],
                    o_vmem.at[pl.ds(j * WINDOW, WINDOW)],
                    sem,
                )
                for j in range(SUBGATHERS)
            ]
            for c in copies:
                c.wait()

        pltpu.emit_pipeline(
            body,
            grid=(n_idx // step_rows,),
            in_specs=[pl.BlockSpec((SUBGATHERS, WINDOW), index_map=lambda i: (i, 0))],
            out_specs=[
                pl.BlockSpec((step_rows, 2 * EMBED_DIM), index_map=lambda i: (i, 0))
            ],
            core_axis_name=("core", "subcore"),
            dimension_semantics=(pltpu.PARALLEL,),
        )(idx_hbm, out_hbm)

    pairs = gather_kernel(table2, rows)
    half = (idx & 1)[:, None]
    out = jnp.where(half == 1, pairs[:, EMBED_DIM:], pairs[:, :EMBED_DIM])
    return out.reshape(batch, hist, EMBED_DIM)


# speedup vs baseline: 1.6063x; 1.6063x over previous
import jax
import jax.numpy as jnp
from jax.experimental import pallas as pl
from jax.experimental.pallas import tpu as pltpu
from jax.experimental.pallas import tpu_sc as plsc

EMBED_DIM = 64
WINDOW = 128
SUBGATHERS = 4


def kernel(token_ids, embedding):
    batch, hist = token_ids.shape
    n_idx = batch * hist
    step_rows = WINDOW * SUBGATHERS

    idx = token_ids.reshape(-1).astype(jnp.int32)
    rows = idx.reshape(n_idx // WINDOW, WINDOW)

    mesh = plsc.VectorSubcoreMesh(core_axis_name="core", subcore_axis_name="subcore")

    @pl.kernel(
        out_type=jax.ShapeDtypeStruct((n_idx, EMBED_DIM), embedding.dtype),
        mesh=mesh,
        scratch_types=[pltpu.SemaphoreType.DMA],
        compiler_params=pltpu.CompilerParams(use_tc_tiling_on_sc=False),
    )
    def gather_kernel(table_hbm, idx_hbm, out_hbm, sem):
        def body(i_vmem, o_vmem):
            copies = [
                pltpu.async_copy(
                    table_hbm.at[i_vmem.at[j]],
                    o_vmem.at[pl.ds(j * WINDOW, WINDOW)],
                    sem,
                )
                for j in range(SUBGATHERS)
            ]
            for c in copies:
                c.wait()

        pltpu.emit_pipeline(
            body,
            grid=(n_idx // step_rows,),
            in_specs=[pl.BlockSpec((SUBGATHERS, WINDOW), index_map=lambda i: (i, 0))],
            out_specs=[
                pl.BlockSpec((step_rows, EMBED_DIM), index_map=lambda i: (i, 0))
            ],
            core_axis_name=("core", "subcore"),
            dimension_semantics=(pltpu.PARALLEL,),
        )(idx_hbm, out_hbm)

    out = gather_kernel(embedding, rows)
    return out.reshape(batch, hist, EMBED_DIM)


# speedup vs baseline: 1.6106x; 1.0026x over previous
import jax
import jax.numpy as jnp
from jax.experimental import pallas as pl
from jax.experimental.pallas import tpu as pltpu
from jax.experimental.pallas import tpu_sc as plsc

EMBED_DIM = 64
WINDOW = 128
SUBGATHERS = 5


def kernel(token_ids, embedding):
    batch, hist = token_ids.shape
    n_idx = batch * hist
    step_rows = WINDOW * SUBGATHERS

    idx = token_ids.reshape(-1).astype(jnp.int32)
    rows = idx.reshape(n_idx // WINDOW, WINDOW)

    mesh = plsc.VectorSubcoreMesh(core_axis_name="core", subcore_axis_name="subcore")

    @pl.kernel(
        out_type=jax.ShapeDtypeStruct((n_idx, EMBED_DIM), embedding.dtype),
        mesh=mesh,
        scratch_types=[pltpu.SemaphoreType.DMA],
        compiler_params=pltpu.CompilerParams(use_tc_tiling_on_sc=False),
    )
    def gather_kernel(table_hbm, idx_hbm, out_hbm, sem):
        def body(i_vmem, o_vmem):
            copies = [
                pltpu.async_copy(
                    table_hbm.at[i_vmem.at[j]],
                    o_vmem.at[pl.ds(j * WINDOW, WINDOW)],
                    sem,
                )
                for j in range(SUBGATHERS)
            ]
            for c in copies:
                c.wait()

        pltpu.emit_pipeline(
            body,
            grid=(n_idx // step_rows,),
            in_specs=[pl.BlockSpec((SUBGATHERS, WINDOW), index_map=lambda i: (i, 0))],
            out_specs=[
                pl.BlockSpec((step_rows, EMBED_DIM), index_map=lambda i: (i, 0))
            ],
            core_axis_name=("core", "subcore"),
            dimension_semantics=(pltpu.PARALLEL,),
        )(idx_hbm, out_hbm)

    out = gather_kernel(embedding, rows)
    return out.reshape(batch, hist, EMBED_DIM)


# speedup vs baseline: 1.6122x; 1.0010x over previous
import jax
import jax.numpy as jnp
from jax import lax
from jax.experimental import pallas as pl
from jax.experimental.pallas import tpu as pltpu
from jax.experimental.pallas import tpu_sc as plsc

EMBED_DIM = 64
WINDOW = 128
DEPTH = 8
NBUF = 10
N_WORKERS = 32


def kernel(token_ids, embedding):
    batch, hist = token_ids.shape
    n_idx = batch * hist
    n_chunks = n_idx // WINDOW
    chunks_per_worker = n_chunks // N_WORKERS
    n_rounds = chunks_per_worker // NBUF

    idx = token_ids.reshape(n_chunks, WINDOW).astype(jnp.int32)

    mesh = plsc.VectorSubcoreMesh(core_axis_name="core", subcore_axis_name="subcore")

    @pl.kernel(
        out_type=jax.ShapeDtypeStruct((n_idx, EMBED_DIM), embedding.dtype),
        mesh=mesh,
        scratch_types=[
            pltpu.VMEM((chunks_per_worker, WINDOW), jnp.int32),
            pltpu.VMEM((NBUF * WINDOW, EMBED_DIM), jnp.float32),
            pltpu.SemaphoreType.DMA((NBUF,)),
            pltpu.SemaphoreType.DMA((NBUF,)),
        ],
        compiler_params=pltpu.CompilerParams(use_tc_tiling_on_sc=False),
    )
    def gather_kernel(table_hbm, idx_hbm, out_hbm, idx_v, rows_v, gsem, wsem):
        wid = lax.axis_index("subcore") * 2 + lax.axis_index("core")
        chunk0 = wid * chunks_per_worker

        def row_buf(s):
            return rows_v.at[pl.ds(s * WINDOW, WINDOW)]

        def start_gather(c, s):
            pltpu.async_copy(table_hbm.at[idx_v.at[c]], row_buf(s), gsem.at[s])

        def wait_gather(s):
            pltpu.make_async_copy(table_hbm.at[idx_v.at[0]], row_buf(s), gsem.at[s]).wait()

        def out_slice(c):
            return out_hbm.at[pl.ds((chunk0 + c) * WINDOW, WINDOW)]

        def start_write(c, s):
            pltpu.async_copy(row_buf(s), out_slice(c), wsem.at[s])

        def wait_write(s):
            pltpu.make_async_copy(out_slice(0), row_buf(s), wsem.at[s]).wait()

        pltpu.sync_copy(idx_hbm.at[pl.ds(wid * chunks_per_worker, chunks_per_worker)], idx_v)

        for s in range(DEPTH):
            start_gather(s, s)

        for j in range(NBUF):
            wait_gather(j)
            start_write(j, j)
            s_n = (j + DEPTH) % NBUF
            if j >= NBUF - DEPTH:
                wait_write(s_n)
            start_gather(j + DEPTH, s_n)

        def round_body(r, _):
            c0 = r * NBUF
            for j in range(NBUF):
                wait_gather(j)
                start_write(c0 + j, j)
                s_n = (j + DEPTH) % NBUF
                wait_write(s_n)
                start_gather(c0 + j + DEPTH, s_n)
            return _

        lax.fori_loop(1, n_rounds - 1, round_body, 0)

        c0 = (n_rounds - 1) * NBUF
        for j in range(NBUF):
            wait_gather(j)
            start_write(c0 + j, j)
            if j < NBUF - DEPTH:
                s_n = j + DEPTH
                wait_write(s_n)
                start_gather(c0 + j + DEPTH, s_n)

        for s in range(NBUF):
            wait_write(s)

    out = gather_kernel(embedding, idx)
    return out.reshape(batch, hist, EMBED_DIM)
